# baseline (speedup 1.0000x reference)
import functools
import math

import jax
import jax.numpy as jnp
from jax import lax
from jax.experimental import pallas as pl
from jax.experimental.pallas import tpu as pltpu
from jax.experimental.pallas import tpu_sc as plsc

S = 64.0
MARGIN = 0.5
_COS_M = math.cos(MARGIN)
_SIN_M = math.sin(MARGIN)

_NC = 2
_NS = 16
_NW = _NC * _NS
_L = 16

_BLOCK_R = 2048


def _sqrt1mt2(t):
    z = jnp.maximum(1.0 - t * t, 0.0)
    s = jnp.full_like(z, 0.5)
    for _ in range(22):
        s = 0.5 * (s + z / s)
    return s


def _sc_fix_body(n_rows, n_cols, lt_hbm, labels_hbm, fixed_hbm,
                 labv, idxv, segbuf, tv, valv, sem):
    cols_per_w = n_rows // _NW
    wid = lax.axis_index("s") * _NC + lax.axis_index("c")
    col0 = wid * cols_per_w
    pltpu.sync_copy(labels_hbm.at[pl.ds(col0, cols_per_w)], labv)
    for h in range(cols_per_w // _L):
        lab = labv[pl.ds(h * _L, _L)]
        idxv[pl.ds(h * _L, _L)] = jnp.where(lab >= 0, lab, 0)
    copies = []
    for k in range(cols_per_w):
        row = idxv[pl.ds((k // _L) * _L, _L)][k % _L]
        seg = ((col0 + k) // 8) * 8
        copies.append(pltpu.async_copy(
            lt_hbm.at[row, pl.ds(seg, 8)], segbuf.at[pl.ds(_L * k, 8)], sem))
    for c in copies:
        c.wait()
    lane = lax.broadcasted_iota(jnp.int32, (_L,), 0)
    for h in range(cols_per_w // _L):
        lab = labv[pl.ds(h * _L, _L)]
        t = jnp.zeros((_L,), jnp.float32)
        for kk in range(_L):
            k = h * _L + kk
            v = segbuf[pl.ds(_L * k, _L)]
            t = jnp.where(lane == kk, v[k % 8], t)
        fixed = (t * _COS_M - _sqrt1mt2(t) * _SIN_M) * S
        valv[pl.ds(h * _L, _L)] = jnp.where(lab >= 0, fixed, t * S)
    pltpu.sync_copy(valv, fixed_hbm.at[pl.ds(col0, cols_per_w)])


def _sc_fixed_values(lt, labels, n_rows, n_cols):
    cols_per_w = n_rows // _NW
    mesh = plsc.VectorSubcoreMesh(
        core_axis_name="c", subcore_axis_name="s",
        num_cores=_NC, num_subcores=_NS)
    return pl.kernel(
        functools.partial(_sc_fix_body, n_rows, n_cols),
        out_type=jax.ShapeDtypeStruct((n_rows,), jnp.float32),
        mesh=mesh,
        scratch_types=[
            pltpu.VMEM((cols_per_w,), jnp.int32),
            pltpu.VMEM((cols_per_w,), jnp.int32),
            pltpu.VMEM((_L * cols_per_w,), jnp.float32),
            pltpu.VMEM((cols_per_w,), jnp.float32),
            pltpu.VMEM((cols_per_w,), jnp.float32),
            pltpu.SemaphoreType.DMA,
        ],
    )(lt, labels)


def _tc_stream_body(lt_ref, lab_ref, fixed_ref, out_ref, *, block_r):
    j = pl.program_id(0)
    x = lt_ref[...]
    r, c = x.shape
    row_ids = lax.broadcasted_iota(jnp.int32, (r, c), 0) + j * block_r
    mask = row_ids == lab_ref[...]
    out_ref[...] = jnp.where(mask, fixed_ref[...], x * S)


def kernel(logits, labels):
    n_rows, n_cols = logits.shape
    lt = logits.T
    fixed = _sc_fixed_values(lt, labels, n_rows, n_cols)
    labels_row = labels.reshape(1, n_rows)
    out_t = pl.pallas_call(
        functools.partial(_tc_stream_body, block_r=_BLOCK_R),
        grid=(pl.cdiv(n_cols, _BLOCK_R),),
        in_specs=[
            pl.BlockSpec((_BLOCK_R, n_rows), lambda j: (j, 0)),
            pl.BlockSpec((1, n_rows), lambda j: (0, 0)),
            pl.BlockSpec((1, n_rows), lambda j: (0, 0)),
        ],
        out_specs=pl.BlockSpec((_BLOCK_R, n_rows), lambda j: (j, 0)),
        out_shape=jax.ShapeDtypeStruct((n_cols, n_rows), jnp.float32),
    )(lt, labels_row, fixed.reshape(1, n_rows))
    return out_t.T


# speedup vs baseline: 1.0047x; 1.0047x over previous
import functools
import math

import jax
import jax.numpy as jnp
from jax import lax
from jax.experimental import pallas as pl
from jax.experimental.pallas import tpu as pltpu
from jax.experimental.pallas import tpu_sc as plsc

S = 64.0
MARGIN = 0.5
_COS_M = math.cos(MARGIN)
_SIN_M = math.sin(MARGIN)

_NC = 2
_NS = 16
_NW = _NC * _NS
_L = 16

_BLOCK_R = 3072


def _sqrt1mt2(t):
    z = jnp.maximum(1.0 - t * t, 0.0)
    s = jnp.full_like(z, 0.5)
    for _ in range(22):
        s = 0.5 * (s + z / s)
    return s


def _sc_fix_body(n_rows, n_cols, lt_hbm, labels_hbm, fixed_hbm,
                 labv, idxv, segbuf, tv, valv, sem):
    cols_per_w = n_rows // _NW
    wid = lax.axis_index("s") * _NC + lax.axis_index("c")
    col0 = wid * cols_per_w
    pltpu.sync_copy(labels_hbm.at[pl.ds(col0, cols_per_w)], labv)
    for h in range(cols_per_w // _L):
        lab = labv[pl.ds(h * _L, _L)]
        idxv[pl.ds(h * _L, _L)] = jnp.where(lab >= 0, lab, 0)
    copies = []
    for k in range(cols_per_w):
        row = idxv[pl.ds((k // _L) * _L, _L)][k % _L]
        seg = ((col0 + k) // 8) * 8
        copies.append(pltpu.async_copy(
            lt_hbm.at[row, pl.ds(seg, 8)], segbuf.at[pl.ds(_L * k, 8)], sem))
    for c in copies:
        c.wait()
    lane = lax.broadcasted_iota(jnp.int32, (_L,), 0)
    for h in range(cols_per_w // _L):
        lab = labv[pl.ds(h * _L, _L)]
        t = jnp.zeros((_L,), jnp.float32)
        for kk in range(_L):
            k = h * _L + kk
            v = segbuf[pl.ds(_L * k, _L)]
            t = jnp.where(lane == kk, v[k % 8], t)
        fixed = (t * _COS_M - _sqrt1mt2(t) * _SIN_M) * S
        valv[pl.ds(h * _L, _L)] = jnp.where(lab >= 0, fixed, t * S)
    pltpu.sync_copy(valv, fixed_hbm.at[pl.ds(col0, cols_per_w)])


def _sc_fixed_values(lt, labels, n_rows, n_cols):
    cols_per_w = n_rows // _NW
    mesh = plsc.VectorSubcoreMesh(
        core_axis_name="c", subcore_axis_name="s",
        num_cores=_NC, num_subcores=_NS)
    return pl.kernel(
        functools.partial(_sc_fix_body, n_rows, n_cols),
        out_type=jax.ShapeDtypeStruct((n_rows,), jnp.float32),
        mesh=mesh,
        scratch_types=[
            pltpu.VMEM((cols_per_w,), jnp.int32),
            pltpu.VMEM((cols_per_w,), jnp.int32),
            pltpu.VMEM((_L * cols_per_w,), jnp.float32),
            pltpu.VMEM((cols_per_w,), jnp.float32),
            pltpu.VMEM((cols_per_w,), jnp.float32),
            pltpu.SemaphoreType.DMA,
        ],
    )(lt, labels)


def _tc_stream_body(lt_ref, lab_ref, fixed_ref, out_ref, *, block_r):
    j = pl.program_id(0)
    x = lt_ref[...]
    r, c = x.shape
    row_ids = lax.broadcasted_iota(jnp.int32, (r, c), 0) + j * block_r
    mask = row_ids == lab_ref[...]
    out_ref[...] = jnp.where(mask, fixed_ref[...], x * S)


def kernel(logits, labels):
    n_rows, n_cols = logits.shape
    lt = logits.T
    fixed = _sc_fixed_values(lt, labels, n_rows, n_cols)
    labels_row = labels.reshape(1, n_rows)
    out_t = pl.pallas_call(
        functools.partial(_tc_stream_body, block_r=_BLOCK_R),
        grid=(pl.cdiv(n_cols, _BLOCK_R),),
        in_specs=[
            pl.BlockSpec((_BLOCK_R, n_rows), lambda j: (j, 0)),
            pl.BlockSpec((1, n_rows), lambda j: (0, 0)),
            pl.BlockSpec((1, n_rows), lambda j: (0, 0)),
        ],
        out_specs=pl.BlockSpec((_BLOCK_R, n_rows), lambda j: (j, 0)),
        out_shape=jax.ShapeDtypeStruct((n_cols, n_rows), jnp.float32),
    )(lt, labels_row, fixed.reshape(1, n_rows))
    return out_t.T


# speedup vs baseline: 1.0058x; 1.0011x over previous
import functools
import math

import jax
import jax.numpy as jnp
from jax import lax
from jax.experimental import pallas as pl
from jax.experimental.pallas import tpu as pltpu
from jax.experimental.pallas import tpu_sc as plsc

S = 64.0
MARGIN = 0.5
_COS_M = math.cos(MARGIN)
_SIN_M = math.sin(MARGIN)

_NC = 2
_NS = 16
_NW = _NC * _NS
_L = 16

_BLOCK_R = 3072


def _sqrt1mt2(t):
    z = jnp.maximum(1.0 - t * t, 0.0)
    s = jnp.full_like(z, 0.5)
    for _ in range(22):
        s = 0.5 * (s + z / s)
    return s


def _sc_fix_body(n_rows, lt_hbm, labels_hbm, fixed_hbm,
                 labv, idxv, segbuf, valv, sem):
    cols_per_w = n_rows // _NW
    wid = lax.axis_index("s") * _NC + lax.axis_index("c")
    col0 = wid * cols_per_w
    pltpu.sync_copy(labels_hbm.at[pl.ds(col0, cols_per_w)], labv)
    for h in range(cols_per_w // _L):
        lab = labv[pl.ds(h * _L, _L)]
        idxv[pl.ds(h * _L, _L)] = jnp.where(lab >= 0, lab, 0)
    copies = []
    for k in range(cols_per_w):
        row = idxv[pl.ds((k // _L) * _L, _L)][k % _L]
        seg = ((col0 + k) // 8) * 8
        copies.append(pltpu.async_copy(
            lt_hbm.at[row, pl.ds(seg, 8)], segbuf.at[pl.ds(_L * k, 8)], sem))
    for c in copies:
        c.wait()
    lane = lax.broadcasted_iota(jnp.int32, (_L,), 0)
    for h in range(cols_per_w // _L):
        lab = labv[pl.ds(h * _L, _L)]
        t = jnp.zeros((_L,), jnp.float32)
        for kk in range(_L):
            k = h * _L + kk
            v = segbuf[pl.ds(_L * k, _L)]
            t = jnp.where(lane == kk, v[k % 8], t)
        fixed = (t * _COS_M - _sqrt1mt2(t) * _SIN_M) * S
        valv[pl.ds(h * _L, _L)] = jnp.where(lab >= 0, fixed, t * S)
    pltpu.sync_copy(valv, fixed_hbm.at[pl.ds(col0, cols_per_w)])


def _sc_fixed_values(lt, labels, n_rows, n_cols):
    cols_per_w = n_rows // _NW
    mesh = plsc.VectorSubcoreMesh(
        core_axis_name="c", subcore_axis_name="s",
        num_cores=_NC, num_subcores=_NS)
    return pl.kernel(
        functools.partial(_sc_fix_body, n_rows),
        out_type=jax.ShapeDtypeStruct((n_rows,), jnp.float32),
        mesh=mesh,
        scratch_types=[
            pltpu.VMEM((cols_per_w,), jnp.int32),
            pltpu.VMEM((cols_per_w,), jnp.int32),
            pltpu.VMEM((_L * cols_per_w,), jnp.float32),
            pltpu.VMEM((cols_per_w,), jnp.float32),
            pltpu.SemaphoreType.DMA,
        ],
    )(lt, labels)


def _tc_stream_body(lt_ref, lab_ref, fixed_ref, out_ref, *, block_r):
    j = pl.program_id(0)
    x = lt_ref[...]
    r, c = x.shape
    row_ids = lax.broadcasted_iota(jnp.int32, (r, c), 0) + j * block_r
    mask = row_ids == lab_ref[...]
    out_ref[...] = jnp.where(mask, fixed_ref[...], x * S)


def kernel(logits, labels):
    n_rows, n_cols = logits.shape
    lt = logits.T
    fixed = _sc_fixed_values(lt, labels, n_rows, n_cols)
    labels_row = labels.reshape(1, n_rows)
    out_t = pl.pallas_call(
        functools.partial(_tc_stream_body, block_r=_BLOCK_R),
        grid=(pl.cdiv(n_cols, _BLOCK_R),),
        in_specs=[
            pl.BlockSpec((_BLOCK_R, n_rows), lambda j: (j, 0)),
            pl.BlockSpec((1, n_rows), lambda j: (0, 0)),
            pl.BlockSpec((1, n_rows), lambda j: (0, 0)),
        ],
        out_specs=pl.BlockSpec((_BLOCK_R, n_rows), lambda j: (j, 0)),
        out_shape=jax.ShapeDtypeStruct((n_cols, n_rows), jnp.float32),
    )(lt, labels_row, fixed.reshape(1, n_rows))
    return out_t.T
